# baseline (speedup 1.0000x reference)
import functools

import jax
import jax.numpy as jnp
from jax import lax
from jax.experimental import pallas as pl
from jax.experimental.pallas import tpu as pltpu
from jax.experimental.pallas import tpu_sc as plsc

B = 128
S = 32768
K = 8
RB = 8

NC = 2
NS = 16
NW = NC * NS
FLAT = B * K
RPW = FLAT // NW
WPR = S // 16


def _stats_body(logits_ref, mask_ref, idx_ref, scores_ref):
    x = logits_ref[...] + (mask_ref[...] - 1.0) * 1e9
    iota = lax.broadcasted_iota(jnp.int32, (RB, S), 1)
    m0 = jnp.max(x, axis=1, keepdims=True)
    ssum = jnp.sum(jnp.exp(x - m0), axis=1, keepdims=True)
    cur = x
    idxs, scs = [], []
    for _ in range(K):
        mj = jnp.max(cur, axis=1, keepdims=True)
        eq = cur == mj
        idxj = jnp.min(jnp.where(eq, iota, S), axis=1, keepdims=True)
        cur = jnp.where(iota == idxj, -3.0e38, cur)
        idxs.append(idxj)
        pj = jnp.exp(mj - m0) / ssum
        scs.append(jnp.log(pj + 1e-20))
    idx_ref[...] = jnp.concatenate(idxs, axis=1)
    scores_ref[...] = jnp.concatenate(scs, axis=1)


def _sc_fill_body(idx_hbm, out_hbm, zbuf_a, zbuf_b, idx_s, sem_a, sem_b):
    wid = lax.axis_index("s") * NC + lax.axis_index("c")
    row0 = wid * RPW
    pltpu.sync_copy(idx_hbm.at[pl.ds(row0, RPW)], idx_s)

    zero16 = jnp.zeros((16,), jnp.float32)
    iota16 = lax.iota(jnp.int32, 16)
    bufs = (zbuf_a, zbuf_b)
    sems = (sem_a, sem_b)

    def _zloop(i, c):
        zbuf_a[pl.ds(i * 16, 16)] = zero16
        zbuf_b[pl.ds(i * 16, 16)] = zero16
        return c

    lax.fori_loop(0, WPR, _zloop, 0)

    ivs = []
    for c in range(RPW // 16):
        iv16 = idx_s[pl.ds(c * 16, 16)]
        for u in range(16):
            ivs.append(iv16[u])

    handles = [None, None]
    prev_w = [None, None]
    for t in range(RPW):
        sb = t % 2
        if handles[sb] is not None:
            handles[sb].wait()
            bufs[sb][pl.ds(prev_w[sb], 16)] = zero16
        w = jnp.right_shift(ivs[t], 4) * 16
        bufs[sb][pl.ds(w, 16)] = jnp.where(
            iota16 == jnp.bitwise_and(ivs[t], 15), 1.0, 0.0
        ).astype(jnp.float32)
        handles[sb] = pltpu.async_copy(
            bufs[sb], out_hbm.at[pl.ds((row0 + t) * S, S)], sems[sb]
        )
        prev_w[sb] = w
    handles[0].wait()
    handles[1].wait()


_sc_fill = functools.partial(
    pl.kernel,
    out_type=jax.ShapeDtypeStruct((FLAT * S,), jnp.float32),
    mesh=plsc.VectorSubcoreMesh(core_axis_name="c", subcore_axis_name="s"),
    scratch_types=[
        pltpu.VMEM((S,), jnp.float32),
        pltpu.VMEM((S,), jnp.float32),
        pltpu.VMEM((RPW,), jnp.int32),
        pltpu.SemaphoreType.DMA,
        pltpu.SemaphoreType.DMA,
    ],
)(_sc_fill_body)


def kernel(logits, mask, k):
    del k
    idx, scores = pl.pallas_call(
        _stats_body,
        grid=(B // RB,),
        in_specs=[
            pl.BlockSpec((RB, S), lambda g: (g, 0)),
            pl.BlockSpec((RB, S), lambda g: (g, 0)),
        ],
        out_specs=[
            pl.BlockSpec((RB, K), lambda g: (g, 0)),
            pl.BlockSpec((RB, K), lambda g: (g, 0)),
        ],
        out_shape=[
            jax.ShapeDtypeStruct((B, K), jnp.int32),
            jax.ShapeDtypeStruct((B, K), jnp.float32),
        ],
    )(logits, mask)
    sel = _sc_fill(idx.reshape(FLAT)).reshape(B, K, S)
    return (sel, scores)


# speedup vs baseline: 1.9769x; 1.9769x over previous
import functools

import jax
import jax.numpy as jnp
from jax import lax
from jax.experimental import pallas as pl
from jax.experimental.pallas import tpu as pltpu
from jax.experimental.pallas import tpu_sc as plsc

B = 128
S = 32768
K = 8
RB = 8

NC = 2
NS = 16
NW = NC * NS
FLAT = B * K
RPW = FLAT // NW
WPR = S // 16


def _stats_body(logits_ref, mask_ref, idx_ref, scores_ref):
    x = logits_ref[...] + (mask_ref[...] - 1.0) * 1e9
    iota = lax.broadcasted_iota(jnp.int32, (RB, S), 1)
    m0 = jnp.max(x, axis=1, keepdims=True)
    ssum = jnp.sum(jnp.exp(x - m0), axis=1, keepdims=True)
    cur = x
    idxs, scs = [], []
    for _ in range(K):
        mj = jnp.max(cur, axis=1, keepdims=True)
        eq = cur == mj
        idxj = jnp.min(jnp.where(eq, iota, S), axis=1, keepdims=True)
        cur = jnp.where(iota == idxj, -3.0e38, cur)
        idxs.append(idxj)
        pj = jnp.exp(mj - m0) / ssum
        scs.append(jnp.log(pj + 1e-20))
    idx_ref[...] = jnp.concatenate(idxs, axis=1)
    scores_ref[...] = jnp.concatenate(scs, axis=1)


SPW = B // NW
PPS = 8
PC = S // PPS


def _sc_fill_body(idx_hbm, out_hbm, zbuf_a, zbuf_b, idx_s, sem_a, sem_b):
    wid = lax.axis_index("s") * NC + lax.axis_index("c")
    pltpu.sync_copy(idx_hbm.at[pl.ds(wid * SPW * K, SPW * K)], idx_s)

    zero16 = jnp.zeros((16,), jnp.float32)
    iota16 = lax.iota(jnp.int32, 16)
    bufs = (zbuf_a, zbuf_b)
    sems = (sem_a, sem_b)

    def _zloop(i, c):
        for j in range(K):
            o = pl.multiple_of(i * 16, 16)
            zbuf_a[j, pl.ds(o, 16)] = zero16
            zbuf_b[j, pl.ds(o, 16)] = zero16
        return c

    lax.fori_loop(0, PC // 16, _zloop, 0)

    ivs = []
    for c in range(SPW * K // 16):
        iv16 = idx_s[pl.ds(c * 16, 16)]
        for u in range(16):
            ivs.append(iv16[u])

    handles = [None, None]
    prev_pat = [[], []]
    for t in range(SPW * PPS):
        sl, p = t // PPS, t % PPS
        b = wid * SPW + sl
        sb = t % 2
        if handles[sb] is not None:
            handles[sb].wait()
            for cond, j, lb in prev_pat[sb]:
                @pl.when(cond)
                def _clear(sb=sb, j=j, lb=lb):
                    bufs[sb][j, pl.ds(lb, 16)] = zero16
        pats = []
        for j in range(K):
            iv = ivs[sl * K + j]
            cond = jnp.right_shift(iv, 12) == p
            lb = pl.multiple_of(jnp.bitwise_and(iv, PC - 16), 16)
            @pl.when(cond)
            def _patch(sb=sb, j=j, lb=lb, iv=iv):
                bufs[sb][j, pl.ds(lb, 16)] = jnp.where(
                    iota16 == jnp.bitwise_and(iv, 15), 1.0, 0.0
                ).astype(jnp.float32)
            pats.append((cond, j, lb))
        handles[sb] = pltpu.async_copy(
            bufs[sb], out_hbm.at[b, :, pl.ds(p * PC, PC)], sems[sb]
        )
        prev_pat[sb] = pats
    handles[0].wait()
    handles[1].wait()


_sc_fill = functools.partial(
    pl.kernel,
    out_type=jax.ShapeDtypeStruct((B, K, S), jnp.float32),
    mesh=plsc.VectorSubcoreMesh(core_axis_name="c", subcore_axis_name="s"),
    scratch_types=[
        pltpu.VMEM((K, PC), jnp.float32),
        pltpu.VMEM((K, PC), jnp.float32),
        pltpu.VMEM((SPW * K,), jnp.int32),
        pltpu.SemaphoreType.DMA,
        pltpu.SemaphoreType.DMA,
    ],
)(_sc_fill_body)


def kernel(logits, mask, k):
    del k
    idx, scores = pl.pallas_call(
        _stats_body,
        grid=(B // RB,),
        in_specs=[
            pl.BlockSpec((RB, S), lambda g: (g, 0)),
            pl.BlockSpec((RB, S), lambda g: (g, 0)),
        ],
        out_specs=[
            pl.BlockSpec((RB, K), lambda g: (g, 0)),
            pl.BlockSpec((RB, K), lambda g: (g, 0)),
        ],
        out_shape=[
            jax.ShapeDtypeStruct((B, K), jnp.int32),
            jax.ShapeDtypeStruct((B, K), jnp.float32),
        ],
    )(logits, mask)
    sel = _sc_fill(idx.reshape(FLAT))
    return (sel, scores)
